# baseline (speedup 1.0000x reference)
import jax
import jax.numpy as jnp
from jax.experimental import pallas as pl
from jax.experimental.pallas import tpu as pltpu

NUM_EXPERTS_ = 8


def _moe_body(x_ref, disp_ref, comb_ref, w_ref, b_ref, out_ref):
    xb = x_ref[...].astype(jnp.bfloat16)
    db = disp_ref[...].astype(jnp.bfloat16)
    cb = comb_ref[...].astype(jnp.bfloat16)
    wb = w_ref[0].astype(jnp.bfloat16)
    xd = jax.lax.dot_general(
        db, xb,
        (((0,), (0,)), ((), ())),
        preferred_element_type=jnp.float32,
    )
    y = jax.lax.dot_general(
        xd.astype(jnp.bfloat16), wb,
        (((1,), (1,)), ((), ())),
        preferred_element_type=jnp.float32,
    )
    y = y + b_ref[...]
    contrib = jnp.dot(cb, y.astype(jnp.bfloat16), preferred_element_type=jnp.float32)

    @pl.when(pl.program_id(0) == 0)
    def _init():
        out_ref[...] = contrib

    @pl.when(pl.program_id(0) != 0)
    def _acc():
        out_ref[...] += contrib


def kernel(x, combine_array, dispatch_mask, W, b):
    B, T, E, I = x.shape
    C = combine_array.shape[-1]
    O = W.shape[0]
    x2 = x.reshape(T, E * I)
    comb2 = combine_array.reshape(T, E * C)
    disp2 = dispatch_mask.reshape(T, E * C)
    w3 = W.reshape(E, O, I)
    b2 = b.reshape(1, O)

    out = pl.pallas_call(
        _moe_body,
        grid=(E,),
        in_specs=[
            pl.BlockSpec((T, I), lambda e: (0, e)),
            pl.BlockSpec((T, C), lambda e: (0, e)),
            pl.BlockSpec((T, C), lambda e: (0, e)),
            pl.BlockSpec((1, O, I), lambda e: (e, 0, 0)),
            pl.BlockSpec((1, O), lambda e: (0, 0)),
        ],
        out_specs=pl.BlockSpec((T, O), lambda e: (0, 0)),
        out_shape=jax.ShapeDtypeStruct((T, O), jnp.float32),
        compiler_params=pltpu.CompilerParams(
            dimension_semantics=("arbitrary",),
        ),
    )(x2, disp2, comb2, w3, b2)
    return out.reshape(B, T, O)


# speedup vs baseline: 2.3586x; 2.3586x over previous
import jax
import jax.numpy as jnp
from jax.experimental import pallas as pl
from jax.experimental.pallas import tpu as pltpu

E_ = 8
TILE_T = 512


def _dispatch_body(x_ref, disp_ref, w_ref, b_ref, y_ref, xd_acc):
    nt = pl.num_programs(0)
    xt = jnp.transpose(x_ref[0].astype(jnp.bfloat16), (1, 0, 2))
    dt = jnp.transpose(disp_ref[0].astype(jnp.bfloat16), (1, 0, 2))

    @pl.when(pl.program_id(0) == 0)
    def _init():
        xd_acc[...] = jnp.zeros_like(xd_acc)

    for e in range(E_):
        xd_acc[e] += jax.lax.dot_general(
            dt[e], xt[e], (((0,), (0,)), ((), ())),
            preferred_element_type=jnp.float32,
        )

    @pl.when(pl.program_id(0) == nt - 1)
    def _finish():
        for e in range(E_):
            y = jax.lax.dot_general(
                xd_acc[e].astype(jnp.bfloat16), w_ref[e].astype(jnp.bfloat16),
                (((1,), (1,)), ((), ())),
                preferred_element_type=jnp.float32,
            )
            y_ref[e] = (y + b_ref[...]).astype(jnp.bfloat16)


def _combine_body(comb_ref, y_ref, out_ref):
    ct = jnp.transpose(comb_ref[0].astype(jnp.bfloat16), (1, 0, 2))
    acc = jnp.zeros(out_ref.shape, jnp.float32)
    for e in range(E_):
        acc += jnp.dot(ct[e], y_ref[e], preferred_element_type=jnp.float32)
    out_ref[...] = acc


def kernel(x, combine_array, dispatch_mask, W, b):
    B, T, E, I = x.shape
    C = combine_array.shape[-1]
    O = W.shape[0]
    nt = T // TILE_T
    w3 = W.reshape(E, O, I)
    b2 = b.reshape(1, O)

    y = pl.pallas_call(
        _dispatch_body,
        grid=(nt,),
        in_specs=[
            pl.BlockSpec((1, TILE_T, E, I), lambda t: (0, t, 0, 0)),
            pl.BlockSpec((1, TILE_T, E, C), lambda t: (0, t, 0, 0)),
            pl.BlockSpec((E, O, I), lambda t: (0, 0, 0)),
            pl.BlockSpec((1, O), lambda t: (0, 0)),
        ],
        out_specs=pl.BlockSpec((E, C, O), lambda t: (0, 0, 0)),
        out_shape=jax.ShapeDtypeStruct((E, C, O), jnp.bfloat16),
        scratch_shapes=[pltpu.VMEM((E, C, I), jnp.float32)],
        compiler_params=pltpu.CompilerParams(
            dimension_semantics=("arbitrary",),
        ),
    )(x, dispatch_mask, w3, b2)

    out = pl.pallas_call(
        _combine_body,
        grid=(nt,),
        in_specs=[
            pl.BlockSpec((1, TILE_T, E, C), lambda t: (0, t, 0, 0)),
            pl.BlockSpec((E, C, O), lambda t: (0, 0, 0)),
        ],
        out_specs=pl.BlockSpec((TILE_T, O), lambda t: (t, 0)),
        out_shape=jax.ShapeDtypeStruct((T, O), jnp.float32),
        compiler_params=pltpu.CompilerParams(
            dimension_semantics=("arbitrary",),
        ),
    )(combine_array, y)
    return out.reshape(B, T, O)


# speedup vs baseline: 2.6539x; 1.1252x over previous
import jax
import jax.numpy as jnp
from jax.experimental import pallas as pl
from jax.experimental.pallas import tpu as pltpu

E_ = 8
TILE_T = 512


def _moe_body(x_ref, disp_ref, comb_ref, w_ref, b_ref, out_ref, xd_acc, y_s):
    p = pl.program_id(0)
    t = pl.program_id(1)
    nt = pl.num_programs(1)

    @pl.when(p == 0)
    def _dispatch_phase():
        xt = jnp.transpose(x_ref[0].astype(jnp.bfloat16), (1, 0, 2))
        dt = jnp.transpose(disp_ref[0].astype(jnp.bfloat16), (1, 0, 2))

        @pl.when(t == 0)
        def _init():
            xd_acc[...] = jnp.zeros_like(xd_acc)

        for e in range(E_):
            xd_acc[e] += jax.lax.dot_general(
                dt[e], xt[e], (((0,), (0,)), ((), ())),
                preferred_element_type=jnp.float32,
            )

        @pl.when(t == nt - 1)
        def _expert_matmul():
            for e in range(E_):
                y = jax.lax.dot_general(
                    xd_acc[e].astype(jnp.bfloat16), w_ref[e],
                    (((1,), (1,)), ((), ())),
                    preferred_element_type=jnp.float32,
                )
                y_s[e] = (y + b_ref[...]).astype(jnp.bfloat16)

    @pl.when(p == 1)
    def _combine_phase():
        ct = jnp.transpose(comb_ref[0].astype(jnp.bfloat16), (1, 0, 2))
        acc = jnp.zeros(out_ref.shape, jnp.float32)
        for e in range(E_):
            acc += jnp.dot(ct[e], y_s[e], preferred_element_type=jnp.float32)
        out_ref[...] = acc


def kernel(x, combine_array, dispatch_mask, W, b):
    B, T, E, I = x.shape
    C = combine_array.shape[-1]
    O = W.shape[0]
    nt = T // TILE_T
    w3 = W.reshape(E, O, I).astype(jnp.bfloat16)
    b2 = b.reshape(1, O)

    out = pl.pallas_call(
        _moe_body,
        grid=(2, nt),
        in_specs=[
            pl.BlockSpec((1, TILE_T, E, I),
                         lambda p, t: (0, jnp.where(p == 0, t, nt - 1), 0, 0)),
            pl.BlockSpec((1, TILE_T, E, C),
                         lambda p, t: (0, jnp.where(p == 0, t, nt - 1), 0, 0)),
            pl.BlockSpec((1, TILE_T, E, C),
                         lambda p, t: (0, jnp.where(p == 0, 0, t), 0, 0)),
            pl.BlockSpec((E, O, I), lambda p, t: (0, 0, 0)),
            pl.BlockSpec((1, O), lambda p, t: (0, 0)),
        ],
        out_specs=pl.BlockSpec((TILE_T, O), lambda p, t: (jnp.where(p == 0, 0, t), 0)),
        out_shape=jax.ShapeDtypeStruct((T, O), jnp.float32),
        scratch_shapes=[
            pltpu.VMEM((E_, 256, 256), jnp.float32),
            pltpu.VMEM((E_, 256, 768), jnp.bfloat16),
        ],
        compiler_params=pltpu.CompilerParams(
            dimension_semantics=("arbitrary", "arbitrary"),
        ),
    )(x, dispatch_mask, combine_array, w3, b2)
    return out.reshape(B, T, O)


# speedup vs baseline: 3.0699x; 1.1568x over previous
import jax
import jax.numpy as jnp
from jax.experimental import pallas as pl
from jax.experimental.pallas import tpu as pltpu

E_ = 8
TILE_T = 512


def _moe_body(x_ref, disp_ref, comb_ref, w_ref, b_ref, out_ref, xd_acc, y_s):
    p = pl.program_id(0)
    t = pl.program_id(1)
    nt = pl.num_programs(1)

    @pl.when(p == 0)
    def _dispatch_phase():
        xt = jnp.transpose(x_ref[0].astype(jnp.bfloat16), (1, 0, 2))
        dt = jnp.transpose(disp_ref[0].astype(jnp.bfloat16), (1, 0, 2))

        @pl.when(t == 0)
        def _init():
            xd_acc[...] = jnp.zeros_like(xd_acc)

        for e in range(E_):
            xd_acc[e] += jax.lax.dot_general(
                dt[e], xt[e], (((0,), (0,)), ((), ())),
                preferred_element_type=jnp.float32,
            )

        @pl.when(t == nt - 1)
        def _expert_matmul():
            for e in range(E_):
                w_e = w_ref[e * 96:(e + 1) * 96, :].astype(jnp.bfloat16)
                w_e = w_e.reshape(768, 256)
                y = jax.lax.dot_general(
                    xd_acc[e].astype(jnp.bfloat16), w_e,
                    (((1,), (1,)), ((), ())),
                    preferred_element_type=jnp.float32,
                )
                y_s[e] = (y + b_ref[...]).astype(jnp.bfloat16)

    @pl.when(p == 1)
    def _combine_phase():
        ct = jnp.transpose(comb_ref[0].astype(jnp.bfloat16), (1, 0, 2))
        acc = jnp.zeros(out_ref.shape, jnp.float32)
        for e in range(E_):
            acc += jnp.dot(ct[e], y_s[e], preferred_element_type=jnp.float32)
        out_ref[...] = acc


def kernel(x, combine_array, dispatch_mask, W, b):
    B, T, E, I = x.shape
    C = combine_array.shape[-1]
    O = W.shape[0]
    nt = T // TILE_T
    b2 = b.reshape(1, O)

    out = pl.pallas_call(
        _moe_body,
        grid=(2, nt),
        in_specs=[
            pl.BlockSpec((1, TILE_T, E, I),
                         lambda p, t: (0, jnp.where(p == 0, t, nt - 1), 0, 0)),
            pl.BlockSpec((1, TILE_T, E, C),
                         lambda p, t: (0, jnp.where(p == 0, t, nt - 1), 0, 0)),
            pl.BlockSpec((1, TILE_T, E, C),
                         lambda p, t: (0, jnp.where(p == 0, 0, t), 0, 0)),
            pl.BlockSpec((O, E * I), lambda p, t: (0, 0)),
            pl.BlockSpec((1, O), lambda p, t: (0, 0)),
        ],
        out_specs=pl.BlockSpec((TILE_T, O), lambda p, t: (jnp.where(p == 0, 0, t), 0)),
        out_shape=jax.ShapeDtypeStruct((T, O), jnp.float32),
        scratch_shapes=[
            pltpu.VMEM((E_, 256, 256), jnp.float32),
            pltpu.VMEM((E_, 256, 768), jnp.bfloat16),
        ],
        compiler_params=pltpu.CompilerParams(
            dimension_semantics=("arbitrary", "arbitrary"),
        ),
    )(x, dispatch_mask, combine_array, W, b2)
    return out.reshape(B, T, O)
